# baseline (speedup 1.0000x reference)
import functools

import jax
import jax.numpy as jnp
from jax import lax
from jax.experimental import pallas as pl
from jax.experimental.pallas import tpu as pltpu
from jax.experimental.pallas import tpu_sc as plsc

NX, NY = 512, 512
VX, VY = 0.25, 0.25
X_OFF = VX / 2.0 + (-50.0)
Y_OFF = VY / 2.0 + (-50.0)
C = 64
P = 16000
MAXPTS = 15
RAW_F = 4
NCELL = NX * NY

KX = 138
NBLK = 16
NOUT = NBLK * C
BIG = float(2.0 ** 30)

BP = 800

NUM_SC = 2
NUM_SUBCORES = 16
NW = NUM_SC * NUM_SUBCORES
CELLS_PER = NCELL // NW
NV = P // 16
CH = 128
MAXCH = P // CH + 1
NTRASH = 64

BT = 2048


def _make_w_full(W, b):
    f32 = jnp.float32
    w_v = W[0:4, :]
    w_cl = W[4:7, :]
    w_eff = (
        w_v
        + jnp.concatenate([w_cl, jnp.zeros((1, C), f32)], axis=0)
        + jnp.concatenate([W[7:9, :], jnp.zeros((2, C), f32)], axis=0)
    )

    def tile15(row):
        return jnp.concatenate(
            [jnp.tile(row, (1, MAXPTS)), jnp.zeros((row.shape[0], C), f32)], axis=1)

    r_vox = jnp.concatenate(
        [jnp.kron(jnp.eye(MAXPTS, dtype=f32), w_eff), jnp.zeros((60, C), f32)],
        axis=1)
    mcl = jnp.concatenate([-w_cl, jnp.zeros((1, C), f32)], axis=0)
    r_mean = jnp.concatenate(
        [jnp.tile(jnp.tile(mcl, (MAXPTS, 1)), (1, MAXPTS)),
         jnp.zeros((60, C), f32)], axis=1)
    r_cx = tile15(-VX * W[7:8, :])
    r_cy = tile15(-VY * W[8:9, :])
    s_ids = jnp.arange(MAXPTS, dtype=jnp.int32)[:, None]
    t_ids = jnp.arange(NBLK, dtype=jnp.int32)[None, :]
    gate = jnp.where(
        (t_ids < MAXPTS) & (t_ids >= s_ids + 1), -BIG,
        jnp.where((t_ids == MAXPTS) & (s_ids == MAXPTS - 1), -BIG, 0.0),
    ).astype(f32)
    r_gate = jnp.kron(gate, jnp.ones((1, C), f32))
    base = (b[None, :] - X_OFF * W[7:8, :] - Y_OFF * W[8:9, :])
    r_one = jnp.concatenate([jnp.tile(base, (1, MAXPTS)), b[None, :]], axis=1)
    return jnp.concatenate([r_vox, r_mean, r_cx, r_cy, r_gate, r_one], axis=0)


def _pfn_body(vox_ref, npv_ref, coords_ref, wf_ref, vf_ref):
    v60 = vox_ref[...]
    npv = npv_ref[...]
    coords = coords_ref[...].astype(jnp.float32)
    pt = (lax.broadcasted_iota(jnp.int32, (1, MAXPTS), 1) + 1
          ).astype(jnp.float32)
    onehot = (npv == pt).astype(jnp.float32)
    x = jnp.concatenate(
        [v60, v60 * (1.0 / npv), coords, onehot,
         jnp.ones((BP, 1), jnp.float32)], axis=1)
    logits = jnp.dot(x, wf_ref[...], preferred_element_type=jnp.float32)
    m = logits[:, 0:128]
    for k in range(1, NOUT // 128):
        m = jnp.maximum(m, logits[:, 128 * k:128 * (k + 1)])
    f = jnp.maximum(jnp.maximum(m[:, 0:C], m[:, C:2 * C]), 0.0)
    vf_ref[...] = jnp.concatenate([f, f], axis=1)


def _pfn_call(vox60, npv2, coordinates, w_full):
    grid = P // BP
    return pl.pallas_call(
        _pfn_body,
        grid=(grid,),
        in_specs=[
            pl.BlockSpec((BP, 60), lambda g: (g, 0)),
            pl.BlockSpec((BP, 1), lambda g: (g, 0)),
            pl.BlockSpec((BP, 2), lambda g: (g, 0)),
            pl.BlockSpec((KX, NOUT), lambda g: (0, 0)),
        ],
        out_specs=pl.BlockSpec((BP, 128), lambda g: (g, 0)),
        out_shape=jax.ShapeDtypeStruct((P, 128), jnp.float32),
    )(vox60, npv2, coordinates, w_full)


def _sc_dedup_body(coords_hbm, slot_hbm, wi_hbm, wc_hbm, cnt_hbm,
                   idx_l, slot_seg, win_ids, win_cells, cnt_l):
    wid = lax.axis_index("s") * NUM_SC + lax.axis_index("c")
    lo = wid * CELLS_PER
    iota = lax.iota(jnp.int32, 16)

    pltpu.sync_copy(coords_hbm, idx_l.at[pl.ds(0, 2 * P)])

    def init_body(i, _):
        slot_seg[pl.ds(i * 16, 16)] = jnp.full((16,), -1, jnp.int32)
        return 0
    lax.fori_loop(0, CELLS_PER // 16, init_body, 0, unroll=4)

    def dedup_body(k, _):
        pos = (k * 16 + iota) * 2
        x = plsc.load_gather(idx_l, [pos])
        y = plsc.load_gather(idx_l, [pos + 1])
        cell = x * NY + y
        plsc.store_scatter(idx_l, [2 * P + k * 16 + iota], cell)
        ids = k * 16 + iota
        loc = cell - lo
        inr = (loc >= 0) & (loc < CELLS_PER)
        locc = jnp.clip(loc, 0, CELLS_PER - 1)
        plsc.store_scatter(slot_seg, [locc], ids, mask=inr)
        cur = plsc.load_gather(slot_seg, [locc], mask=inr)
        want = jnp.where(inr & (cur < ids), 1, 0)

        def cond(w):
            return jnp.max(w) > 0

        def retry(w):
            wb = w > 0
            plsc.store_scatter(slot_seg, [locc], ids, mask=wb)
            cur2 = plsc.load_gather(slot_seg, [locc], mask=wb)
            return jnp.where(wb & (cur2 < ids), 1, 0)

        lax.while_loop(cond, retry, want)
        return 0
    lax.fori_loop(0, NV, dedup_body, 0, unroll=2)

    def comp_body(k, cnt):
        cell = idx_l[pl.ds(2 * P + k * 16, 16)]
        ids = k * 16 + iota
        loc = cell - lo
        inr = (loc >= 0) & (loc < CELLS_PER)
        locc = jnp.clip(loc, 0, CELLS_PER - 1)
        cur = plsc.load_gather(slot_seg, [locc], mask=inr)
        win = inr & (cur == ids)
        m = jnp.where(win, 1, 0)
        pos = cnt + plsc.cumsum(m) - m
        row = lax.shift_right_logical(pos, 7)
        col = pos & (CH - 1)
        plsc.store_scatter(win_ids, [row, col], ids, mask=win)
        plsc.store_scatter(win_cells, [row, col], cell, mask=win)
        return cnt + plsc.all_reduce_population_count(win)

    cntv = lax.fori_loop(0, NV, comp_body, jnp.zeros((16,), jnp.int32),
                         unroll=2)
    total = jnp.max(cntv)

    def pad_body(j, _):
        pos = total + j * 16 + iota
        row = lax.shift_right_logical(pos, 7)
        col = pos & (CH - 1)
        plsc.store_scatter(win_ids, [row, col], pos & (NTRASH - 1))
        plsc.store_scatter(win_cells, [row, col], NCELL + (pos & (NTRASH - 1)))
        return 0
    lax.fori_loop(0, CH // 16, pad_body, 0)

    cnt_l[...] = cntv
    pltpu.sync_copy(slot_seg, slot_hbm.at[pl.ds(lo, CELLS_PER)])
    pltpu.sync_copy(win_ids, wi_hbm.at[wid])
    pltpu.sync_copy(win_cells, wc_hbm.at[wid])
    pltpu.sync_copy(cnt_l, cnt_hbm.at[wid])


def _sc_move_body(vf_hbm, wi_hbm, wc_hbm, cnt_hbm, ct_hbm,
                  win_ids, win_cells, cnt_l, rows, sem_g, sem_s):
    wid = lax.axis_index("s") * NUM_SC + lax.axis_index("c")
    pltpu.sync_copy(wi_hbm.at[wid], win_ids)
    pltpu.sync_copy(wc_hbm.at[wid], win_cells)
    pltpu.sync_copy(cnt_hbm.at[wid], cnt_l)
    total = jnp.max(cnt_l[...])

    def ch_body(ci, _):
        pltpu.async_copy(vf_hbm.at[win_ids.at[ci]], rows, sem_g).wait()
        pltpu.async_copy(rows, ct_hbm.at[win_cells.at[ci]], sem_s).wait()
        return 0
    nch = (total + CH - 1) // CH
    lax.fori_loop(0, nch, ch_body, 0)


def _sc_mesh():
    return plsc.VectorSubcoreMesh(
        core_axis_name="c", subcore_axis_name="s",
        num_cores=NUM_SC, num_subcores=NUM_SUBCORES,
    )


@functools.cache
def _sc_dedup_call():
    return pl.kernel(
        _sc_dedup_body,
        out_type=(
            jax.ShapeDtypeStruct((NCELL,), jnp.int32),
            jax.ShapeDtypeStruct((NW, MAXCH, CH), jnp.int32),
            jax.ShapeDtypeStruct((NW, MAXCH, CH), jnp.int32),
            jax.ShapeDtypeStruct((NW, 16), jnp.int32),
        ),
        mesh=_sc_mesh(),
        scratch_types=[
            pltpu.VMEM((3 * P,), jnp.int32),
            pltpu.VMEM((CELLS_PER,), jnp.int32),
            pltpu.VMEM((MAXCH, CH), jnp.int32),
            pltpu.VMEM((MAXCH, CH), jnp.int32),
            pltpu.VMEM((16,), jnp.int32),
        ],
        compiler_params=pltpu.CompilerParams(
            needs_layout_passes=False, use_tc_tiling_on_sc=True),
    )


@functools.cache
def _sc_move_call():
    return pl.kernel(
        _sc_move_body,
        out_type=jax.ShapeDtypeStruct((NCELL + NTRASH, 128), jnp.float32),
        mesh=_sc_mesh(),
        scratch_types=[
            pltpu.VMEM((MAXCH, CH), jnp.int32),
            pltpu.VMEM((MAXCH, CH), jnp.int32),
            pltpu.VMEM((16,), jnp.int32),
            pltpu.VMEM((CH, 128), jnp.float32),
            pltpu.SemaphoreType.DMA,
            pltpu.SemaphoreType.DMA,
        ],
        compiler_params=pltpu.CompilerParams(
            needs_layout_passes=False, use_tc_tiling_on_sc=True),
    )


def _xpose_body(ct_ref, slot_ref, out_ref):
    a = ct_ref[...]
    s = slot_ref[...].reshape(1, BT)
    out_ref[...] = jnp.where(s >= 0, a[:, 0:C].T, 0.0)


def _xpose_call(ct, slot3):
    grid = NCELL // BT
    return pl.pallas_call(
        _xpose_body,
        grid=(grid,),
        in_specs=[
            pl.BlockSpec((BT, 128), lambda g: (g, 0)),
            pl.BlockSpec((1, 1, BT), lambda g: (g, 0, 0)),
        ],
        out_specs=pl.BlockSpec((C, BT), lambda g: (0, g)),
        out_shape=jax.ShapeDtypeStruct((C, NCELL), jnp.float32),
    )(ct, slot3)


def kernel(voxels, num_points_per_voxel, coordinates, W, b):
    vox60 = voxels.reshape(P, MAXPTS * RAW_F)
    npv2 = num_points_per_voxel.astype(jnp.float32).reshape(P, 1)
    w_full = _make_w_full(W, b)
    vf = _pfn_call(vox60, npv2, coordinates, w_full)
    slot, wi, wc, cnt = _sc_dedup_call()(coordinates.reshape(2 * P))
    ct = _sc_move_call()(vf, wi, wc, cnt)
    return _xpose_call(ct, slot.reshape(NCELL // BT, 1, BT))


# speedup vs baseline: 1.1576x; 1.1576x over previous
import functools

import jax
import jax.numpy as jnp
from jax import lax
from jax.experimental import pallas as pl
from jax.experimental.pallas import tpu as pltpu
from jax.experimental.pallas import tpu_sc as plsc

NX, NY = 512, 512
VX, VY = 0.25, 0.25
X_OFF = VX / 2.0 + (-50.0)
Y_OFF = VY / 2.0 + (-50.0)
C = 64
P = 16000
MAXPTS = 15
RAW_F = 4
NCELL = NX * NY

KX = 138
NBLK = 16
NOUT = NBLK * C
BIG = float(2.0 ** 30)

BP = 800

NUM_SC = 2
NUM_SUBCORES = 16
NW = NUM_SC * NUM_SUBCORES
CELLS_PER = NCELL // NW
NV = P // 16
CH = 128
MAXCH = P // CH + 1
NTRASH = 64

BT = 4096


def _make_w_full(W, b):
    f32 = jnp.float32
    w_v = W[0:4, :]
    w_cl = W[4:7, :]
    w_eff = (
        w_v
        + jnp.concatenate([w_cl, jnp.zeros((1, C), f32)], axis=0)
        + jnp.concatenate([W[7:9, :], jnp.zeros((2, C), f32)], axis=0)
    )

    def tile15(row):
        return jnp.concatenate(
            [jnp.tile(row, (1, MAXPTS)), jnp.zeros((row.shape[0], C), f32)], axis=1)

    r_vox = jnp.concatenate(
        [jnp.kron(jnp.eye(MAXPTS, dtype=f32), w_eff), jnp.zeros((60, C), f32)],
        axis=1)
    mcl = jnp.concatenate([-w_cl, jnp.zeros((1, C), f32)], axis=0)
    r_mean = jnp.concatenate(
        [jnp.tile(jnp.tile(mcl, (MAXPTS, 1)), (1, MAXPTS)),
         jnp.zeros((60, C), f32)], axis=1)
    r_cx = tile15(-VX * W[7:8, :])
    r_cy = tile15(-VY * W[8:9, :])
    s_ids = jnp.arange(MAXPTS, dtype=jnp.int32)[:, None]
    t_ids = jnp.arange(NBLK, dtype=jnp.int32)[None, :]
    gate = jnp.where(
        (t_ids < MAXPTS) & (t_ids >= s_ids + 1), -BIG,
        jnp.where((t_ids == MAXPTS) & (s_ids == MAXPTS - 1), -BIG, 0.0),
    ).astype(f32)
    r_gate = jnp.kron(gate, jnp.ones((1, C), f32))
    base = (b[None, :] - X_OFF * W[7:8, :] - Y_OFF * W[8:9, :])
    r_one = jnp.concatenate([jnp.tile(base, (1, MAXPTS)), b[None, :]], axis=1)
    return jnp.concatenate([r_vox, r_mean, r_cx, r_cy, r_gate, r_one], axis=0)


def _pfn_body(vox_ref, npv_ref, coords_ref, wf_ref, vf_ref):
    v60 = vox_ref[...]
    npv = npv_ref[...]
    coords = coords_ref[...].astype(jnp.float32)
    pt = (lax.broadcasted_iota(jnp.int32, (1, MAXPTS), 1) + 1
          ).astype(jnp.float32)
    onehot = (npv == pt).astype(jnp.float32)
    x = jnp.concatenate(
        [v60, v60 * (1.0 / npv), coords, onehot,
         jnp.ones((BP, 1), jnp.float32)], axis=1)
    logits = jnp.dot(x, wf_ref[...], preferred_element_type=jnp.float32)
    m = logits[:, 0:128]
    for k in range(1, NOUT // 128):
        m = jnp.maximum(m, logits[:, 128 * k:128 * (k + 1)])
    f = jnp.maximum(jnp.maximum(m[:, 0:C], m[:, C:2 * C]), 0.0)
    vf_ref[...] = jnp.concatenate([f, f], axis=1)


def _pfn_call(vox60, npv2, coordinates, w_full):
    grid = P // BP
    return pl.pallas_call(
        _pfn_body,
        grid=(grid,),
        in_specs=[
            pl.BlockSpec((BP, 60), lambda g: (g, 0)),
            pl.BlockSpec((BP, 1), lambda g: (g, 0)),
            pl.BlockSpec((BP, 2), lambda g: (g, 0)),
            pl.BlockSpec((KX, NOUT), lambda g: (0, 0)),
        ],
        out_specs=pl.BlockSpec((BP, 128), lambda g: (g, 0)),
        out_shape=jax.ShapeDtypeStruct((P, 128), jnp.float32),
    )(vox60, npv2, coordinates, w_full)


def _sc_dedup_body(coords_hbm, slot_hbm, wi_hbm, wc_hbm, cnt_hbm,
                   idx_l, slot_seg, win_ids, win_cells, cnt_l):
    wid = lax.axis_index("s") * NUM_SC + lax.axis_index("c")
    lo = wid * CELLS_PER
    iota = lax.iota(jnp.int32, 16)

    pltpu.sync_copy(coords_hbm, idx_l.at[pl.ds(0, 2 * P)])

    def init_body(i, _):
        slot_seg[pl.ds(i * 16, 16)] = jnp.full((16,), -1, jnp.int32)
        return 0
    lax.fori_loop(0, CELLS_PER // 16, init_body, 0, unroll=4)

    def dedup_body(k, _):
        pos = (k * 16 + iota) * 2
        x = plsc.load_gather(idx_l, [pos])
        y = plsc.load_gather(idx_l, [pos + 1])
        cell = x * NY + y
        plsc.store_scatter(idx_l, [2 * P + k * 16 + iota], cell)
        ids = k * 16 + iota
        loc = cell - lo
        inr = (loc >= 0) & (loc < CELLS_PER)
        locc = jnp.clip(loc, 0, CELLS_PER - 1)
        plsc.store_scatter(slot_seg, [locc], ids, mask=inr)
        cur = plsc.load_gather(slot_seg, [locc], mask=inr)
        want = jnp.where(inr & (cur < ids), 1, 0)

        def cond(w):
            return jnp.max(w) > 0

        def retry(w):
            wb = w > 0
            plsc.store_scatter(slot_seg, [locc], ids, mask=wb)
            cur2 = plsc.load_gather(slot_seg, [locc], mask=wb)
            return jnp.where(wb & (cur2 < ids), 1, 0)

        lax.while_loop(cond, retry, want)
        return 0
    lax.fori_loop(0, NV, dedup_body, 0, unroll=2)

    def comp_body(k, cnt):
        cell = idx_l[pl.ds(2 * P + k * 16, 16)]
        ids = k * 16 + iota
        loc = cell - lo
        inr = (loc >= 0) & (loc < CELLS_PER)
        locc = jnp.clip(loc, 0, CELLS_PER - 1)
        cur = plsc.load_gather(slot_seg, [locc], mask=inr)
        win = inr & (cur == ids)
        m = jnp.where(win, 1, 0)
        pos = cnt + plsc.cumsum(m) - m
        row = lax.shift_right_logical(pos, 7)
        col = pos & (CH - 1)
        plsc.store_scatter(win_ids, [row, col], ids, mask=win)
        plsc.store_scatter(win_cells, [row, col], cell, mask=win)
        return cnt + plsc.all_reduce_population_count(win)

    cntv = lax.fori_loop(0, NV, comp_body, jnp.zeros((16,), jnp.int32),
                         unroll=2)
    total = jnp.max(cntv)

    def pad_body(j, _):
        pos = total + j * 16 + iota
        row = lax.shift_right_logical(pos, 7)
        col = pos & (CH - 1)
        plsc.store_scatter(win_ids, [row, col], pos & (NTRASH - 1))
        plsc.store_scatter(win_cells, [row, col], NCELL + (pos & (NTRASH - 1)))
        return 0
    lax.fori_loop(0, CH // 16, pad_body, 0)

    cnt_l[...] = cntv
    pltpu.sync_copy(slot_seg, slot_hbm.at[pl.ds(lo, CELLS_PER)])
    pltpu.sync_copy(win_ids, wi_hbm.at[wid])
    pltpu.sync_copy(win_cells, wc_hbm.at[wid])
    pltpu.sync_copy(cnt_l, cnt_hbm.at[wid])


def _sc_move_body(vf_hbm, wi_hbm, wc_hbm, cnt_hbm, ct_hbm,
                  win_ids, win_cells, cnt_l, rows, sem_g, sem_s):
    wid = lax.axis_index("s") * NUM_SC + lax.axis_index("c")
    pltpu.sync_copy(wi_hbm.at[wid], win_ids)
    pltpu.sync_copy(wc_hbm.at[wid], win_cells)
    pltpu.sync_copy(cnt_hbm.at[wid], cnt_l)
    total = jnp.max(cnt_l[...])

    def ch_body(ci, _):
        pltpu.async_copy(vf_hbm.at[win_ids.at[ci]], rows, sem_g).wait()
        pltpu.async_copy(rows, ct_hbm.at[win_cells.at[ci]], sem_s).wait()
        return 0
    nch = (total + CH - 1) // CH
    lax.fori_loop(0, nch, ch_body, 0)


def _sc_mesh():
    return plsc.VectorSubcoreMesh(
        core_axis_name="c", subcore_axis_name="s",
        num_cores=NUM_SC, num_subcores=NUM_SUBCORES,
    )


@functools.cache
def _sc_dedup_call():
    return pl.kernel(
        _sc_dedup_body,
        out_type=(
            jax.ShapeDtypeStruct((NCELL,), jnp.int32),
            jax.ShapeDtypeStruct((NW, MAXCH, CH), jnp.int32),
            jax.ShapeDtypeStruct((NW, MAXCH, CH), jnp.int32),
            jax.ShapeDtypeStruct((NW, 16), jnp.int32),
        ),
        mesh=_sc_mesh(),
        scratch_types=[
            pltpu.VMEM((3 * P,), jnp.int32),
            pltpu.VMEM((CELLS_PER,), jnp.int32),
            pltpu.VMEM((MAXCH, CH), jnp.int32),
            pltpu.VMEM((MAXCH, CH), jnp.int32),
            pltpu.VMEM((16,), jnp.int32),
        ],
        compiler_params=pltpu.CompilerParams(
            needs_layout_passes=False, use_tc_tiling_on_sc=True),
    )


@functools.cache
def _sc_move_call():
    return pl.kernel(
        _sc_move_body,
        out_type=jax.ShapeDtypeStruct((NCELL + NTRASH, 128), jnp.float32),
        mesh=_sc_mesh(),
        scratch_types=[
            pltpu.VMEM((MAXCH, CH), jnp.int32),
            pltpu.VMEM((MAXCH, CH), jnp.int32),
            pltpu.VMEM((16,), jnp.int32),
            pltpu.VMEM((CH, 128), jnp.float32),
            pltpu.SemaphoreType.DMA,
            pltpu.SemaphoreType.DMA,
        ],
        compiler_params=pltpu.CompilerParams(
            needs_layout_passes=False, use_tc_tiling_on_sc=True),
    )


def _xpose_body(ct_ref, slot_ref, out_ref):
    a = ct_ref[...]
    s = slot_ref[...].reshape(1, BT)
    out_ref[...] = jnp.where(s >= 0, a[:, 0:C].T, 0.0)


def _xpose_call(ct, slot3):
    grid = NCELL // BT
    return pl.pallas_call(
        _xpose_body,
        grid=(grid,),
        in_specs=[
            pl.BlockSpec((BT, 128), lambda g: (g, 0)),
            pl.BlockSpec((1, 1, BT), lambda g: (g, 0, 0)),
        ],
        out_specs=pl.BlockSpec((C, BT), lambda g: (0, g)),
        out_shape=jax.ShapeDtypeStruct((C, NCELL), jnp.float32),
    )(ct, slot3)


def kernel(voxels, num_points_per_voxel, coordinates, W, b):
    vox60 = voxels.reshape(P, MAXPTS * RAW_F)
    npv2 = num_points_per_voxel.astype(jnp.float32).reshape(P, 1)
    w_full = _make_w_full(W, b)
    vf = _pfn_call(vox60, npv2, coordinates, w_full)
    slot, wi, wc, cnt = _sc_dedup_call()(coordinates.reshape(2 * P))
    ct = _sc_move_call()(vf, wi, wc, cnt)
    return _xpose_call(ct, slot.reshape(NCELL // BT, 1, BT))
